# baseline (speedup 1.0000x reference)
import jax
import jax.numpy as jnp
from jax import lax
from jax.experimental import pallas as pl
from jax.experimental.pallas import tpu as pltpu
from jax.experimental.pallas import tpu_sc as plsc

EMBED = 64
LANES = 16
POS_ROWS = 512
NC, NS = 2, 16
NW = NC * NS
SEQ = 200
G1 = 128
G2 = SEQ - G1
NBUF = 4

_mesh = plsc.VectorSubcoreMesh(
    core_axis_name="c", subcore_axis_name="s", num_cores=NC, num_subcores=NS
)


def _body(rel_idx, pos_idx, rel_tab, pos_tab, out,
          pos_v, idx2, pidx2, rows2, sem_idx, sem_g, sem_out):
    n = rel_idx.shape[0]
    nb = out.shape[0]
    b_per_w = nb // NW
    wid = lax.axis_index("s") * NC + lax.axis_index("c")
    b0 = wid * b_per_w

    def fire_idx(i, slot):
        base = (b0 + i) * SEQ
        pltpu.async_copy(rel_idx.at[pl.ds(base, SEQ)],
                         idx2.at[slot, pl.ds(0, SEQ)], sem_idx.at[slot])
        pltpu.async_copy(pos_idx.at[pl.ds(base, SEQ)],
                         pidx2.at[slot, pl.ds(0, SEQ)], sem_idx.at[slot])

    def wait_idx(slot):
        pltpu.make_async_copy(rel_idx.at[pl.ds(0, SEQ)],
                              idx2.at[slot, pl.ds(0, SEQ)], sem_idx.at[slot]).wait()
        pltpu.make_async_copy(pos_idx.at[pl.ds(0, SEQ)],
                              pidx2.at[slot, pl.ds(0, SEQ)], sem_idx.at[slot]).wait()

    def fire_gather(slot):
        pltpu.async_copy(rel_tab.at[idx2.at[slot, pl.ds(0, G1)]],
                         rows2.at[slot, pl.ds(0, G1)], sem_g.at[slot])
        pltpu.async_copy(rel_tab.at[idx2.at[slot, pl.ds(G1, G2)]],
                         rows2.at[slot, pl.ds(G1, G2)], sem_g.at[slot])

    def wait_gather(slot):
        pltpu.make_async_copy(rel_tab.at[idx2.at[slot, pl.ds(0, G1)]],
                              rows2.at[slot, pl.ds(0, G1)], sem_g.at[slot]).wait()
        pltpu.make_async_copy(rel_tab.at[idx2.at[slot, pl.ds(G1, G2)]],
                              rows2.at[slot, pl.ds(G1, G2)], sem_g.at[slot]).wait()

    def fire_out(i, slot):
        pltpu.async_copy(rows2.at[slot], out.at[b0 + i], sem_out.at[slot])

    def wait_out(i, slot):
        pltpu.make_async_copy(rows2.at[slot], out.at[b0 + i], sem_out.at[slot]).wait()

    pltpu.sync_copy(pos_tab, pos_v)

    for slot in range(NBUF):
        pidx2[slot, pl.ds((SEQ // LANES) * LANES, LANES)] = jnp.zeros(
            (LANES,), jnp.int32)

    fire_idx(0, 0)
    fire_idx(1, 1)
    fire_idx(2, 2)
    wait_idx(0)
    fire_gather(0)
    wait_idx(1)
    fire_gather(1)

    iota = lax.iota(jnp.int32, LANES)

    def step(i, carry):
        s = lax.rem(i, NBUF)
        wait_gather(s)

        @pl.when(i + 3 < b_per_w)
        def _():
            fire_idx(i + 3, lax.rem(i + 3, NBUF))

        @pl.when(i + 2 < b_per_w)
        def _():
            s2 = lax.rem(i + 2, NBUF)

            @pl.when(i >= 2)
            def _():
                wait_out(i - 2, s2)

            wait_idx(s2)
            fire_gather(s2)

        rr = rows2.at[s]
        pp = pidx2.at[s]

        def grp(g, gcarry):
            r0 = g * LANES
            pvec = pp[pl.ds(r0, LANES)]
            row_ids = iota + r0
            for col in range(EMBED):
                cvec = jnp.full((LANES,), col, jnp.int32)
                v = plsc.load_gather(pos_v, [pvec, cvec])
                plsc.addupdate_scatter(rr, [row_ids, cvec], v)
            return gcarry

        lax.fori_loop(0, SEQ // LANES, grp, 0)

        r0t = (SEQ // LANES) * LANES
        tmask = iota < (SEQ - r0t)
        pvec_t = pp[pl.ds(r0t, LANES)]
        row_ids_t = iota + r0t
        for col in range(EMBED):
            cvec = jnp.full((LANES,), col, jnp.int32)
            v = plsc.load_gather(pos_v, [pvec_t, cvec], mask=tmask)
            plsc.addupdate_scatter(rr, [row_ids_t, cvec], v, mask=tmask)

        fire_out(i, s)
        return carry

    lax.fori_loop(0, b_per_w, step, 0)
    for j in range(NBUF):
        i = b_per_w - NBUF + j
        wait_out(i, lax.rem(i, NBUF))


def kernel(rel_seq, pos_seq, rel_table, pos_table):
    b, s = rel_seq.shape
    n = b * s
    run = pl.kernel(
        _body,
        out_type=jax.ShapeDtypeStruct((b, s, EMBED), jnp.float32),
        mesh=_mesh,
        scratch_types=[
            pltpu.VMEM((POS_ROWS, EMBED), jnp.float32),
            pltpu.VMEM((NBUF, 256), jnp.int32),
            pltpu.VMEM((NBUF, 256), jnp.int32),
            pltpu.VMEM((NBUF, SEQ, EMBED), jnp.float32),
            pltpu.SemaphoreType.DMA((NBUF,)),
            pltpu.SemaphoreType.DMA((NBUF,)),
            pltpu.SemaphoreType.DMA((NBUF,)),
        ],
        compiler_params=pltpu.CompilerParams(
            use_tc_tiling_on_sc=False, needs_layout_passes=False
        ),
    )
    return run(rel_seq.reshape(n), pos_seq.reshape(n), rel_table, pos_table)


# speedup vs baseline: 1.8919x; 1.8919x over previous
import jax
import jax.numpy as jnp
from jax import lax
from jax.experimental import pallas as pl
from jax.experimental.pallas import tpu as pltpu
from jax.experimental.pallas import tpu_sc as plsc

EMBED = 64
LANES = 16
POS_ROWS = 512
NC, NS = 2, 16
NW = NC * NS
SEQ = 200
GW = 40
NBUF = 4

_mesh = plsc.VectorSubcoreMesh(
    core_axis_name="c", subcore_axis_name="s", num_cores=NC, num_subcores=NS
)


def _body(rel_idx, pos_idx, rel_tab, pos_tab, out,
          pos_v, idx2, pidx2, rows2, sem_idx, sem_g, sem_out):
    n = rel_idx.shape[0]
    nb = out.shape[0]
    b_per_w = nb // NW
    wid = lax.axis_index("s") * NC + lax.axis_index("c")
    b0 = wid * b_per_w

    def fire_idx(i, slot):
        base = (b0 + i) * SEQ
        pltpu.async_copy(rel_idx.at[pl.ds(base, SEQ)],
                         idx2.at[slot, pl.ds(0, SEQ)], sem_idx.at[slot])
        pltpu.async_copy(pos_idx.at[pl.ds(base, SEQ)],
                         pidx2.at[slot, pl.ds(0, SEQ)], sem_idx.at[slot])

    def wait_idx(slot):
        pltpu.make_async_copy(rel_idx.at[pl.ds(0, SEQ)],
                              idx2.at[slot, pl.ds(0, SEQ)], sem_idx.at[slot]).wait()
        pltpu.make_async_copy(pos_idx.at[pl.ds(0, SEQ)],
                              pidx2.at[slot, pl.ds(0, SEQ)], sem_idx.at[slot]).wait()

    def fire_gather(slot):
        for o in range(0, SEQ, GW):
            pltpu.async_copy(rel_tab.at[idx2.at[slot, pl.ds(o, GW)]],
                             rows2.at[slot, pl.ds(o, GW)], sem_g.at[slot])

    def wait_gather(slot):
        for o in range(0, SEQ, GW):
            pltpu.make_async_copy(rel_tab.at[idx2.at[slot, pl.ds(o, GW)]],
                                  rows2.at[slot, pl.ds(o, GW)], sem_g.at[slot]).wait()

    def fire_out(i, slot):
        pltpu.async_copy(rows2.at[slot], out.at[b0 + i], sem_out.at[slot])

    def wait_out(i, slot):
        pltpu.make_async_copy(rows2.at[slot], out.at[b0 + i], sem_out.at[slot]).wait()

    pltpu.sync_copy(pos_tab, pos_v)

    for slot in range(NBUF):
        pidx2[slot, pl.ds((SEQ // LANES) * LANES, LANES)] = jnp.zeros(
            (LANES,), jnp.int32)

    fire_idx(0, 0)
    fire_idx(1, 1)
    fire_idx(2, 2)
    wait_idx(0)
    fire_gather(0)
    wait_idx(1)
    fire_gather(1)

    iota = lax.iota(jnp.int32, LANES)

    def step(i, carry):
        s = lax.rem(i, NBUF)
        wait_gather(s)

        @pl.when(i + 3 < b_per_w)
        def _():
            fire_idx(i + 3, lax.rem(i + 3, NBUF))

        @pl.when(i + 2 < b_per_w)
        def _():
            s2 = lax.rem(i + 2, NBUF)

            @pl.when(i >= 2)
            def _():
                wait_out(i - 2, s2)

            wait_idx(s2)
            fire_gather(s2)

        rr = rows2.at[s]
        pp = pidx2.at[s]

        def row(r, rcarry):
            pb = plsc.load_gather(pp, [jnp.full((LANES,), r, jnp.int32)])
            for c in range(EMBED // LANES):
                v = plsc.load_gather(pos_v, [pb, iota + (c * LANES)])
                plsc.addupdate(rr.at[r, pl.ds(c * LANES, LANES)], v)
            return rcarry

        lax.fori_loop(0, SEQ, row, 0, unroll=2)
        fire_out(i, s)
        return carry

    lax.fori_loop(0, b_per_w, step, 0)
    for j in range(NBUF):
        i = b_per_w - NBUF + j
        wait_out(i, lax.rem(i, NBUF))


def kernel(rel_seq, pos_seq, rel_table, pos_table):
    b, s = rel_seq.shape
    n = b * s
    run = pl.kernel(
        _body,
        out_type=jax.ShapeDtypeStruct((b, s, EMBED), jnp.float32),
        mesh=_mesh,
        scratch_types=[
            pltpu.VMEM((POS_ROWS, EMBED), jnp.float32),
            pltpu.VMEM((NBUF, 256), jnp.int32),
            pltpu.VMEM((NBUF, 256), jnp.int32),
            pltpu.VMEM((NBUF, SEQ, EMBED), jnp.float32),
            pltpu.SemaphoreType.DMA((NBUF,)),
            pltpu.SemaphoreType.DMA((NBUF,)),
            pltpu.SemaphoreType.DMA((NBUF,)),
        ],
        compiler_params=pltpu.CompilerParams(
            use_tc_tiling_on_sc=False, needs_layout_passes=False
        ),
    )
    return run(rel_seq.reshape(n), pos_seq.reshape(n), rel_table, pos_table)


# speedup vs baseline: 2.1741x; 1.1492x over previous
import jax
import jax.numpy as jnp
from jax import lax
from jax.experimental import pallas as pl
from jax.experimental.pallas import tpu as pltpu
from jax.experimental.pallas import tpu_sc as plsc

EMBED = 64
LANES = 16
POS_ROWS = 512
NC, NS = 2, 16
NW = NC * NS
SEQ = 200
GW = 40
NBUF = 4

_mesh = plsc.VectorSubcoreMesh(
    core_axis_name="c", subcore_axis_name="s", num_cores=NC, num_subcores=NS
)


def _body(rel_idx, pos_idx, rel_tab, pos_tab, out,
          pos_v, idx2, pidx2, rows2, sem_idx, sem_g, sem_out):
    n = rel_idx.shape[0]
    nb = out.shape[0]
    b_per_w = nb // NW
    wid = lax.axis_index("s") * NC + lax.axis_index("c")
    b0 = wid * b_per_w

    def fire_idx(i, slot):
        base = (b0 + i) * SEQ
        pltpu.async_copy(rel_idx.at[pl.ds(base, SEQ)],
                         idx2.at[slot, pl.ds(0, SEQ)], sem_idx.at[slot])
        pltpu.async_copy(pos_idx.at[pl.ds(base, SEQ)],
                         pidx2.at[slot, pl.ds(0, SEQ)], sem_idx.at[slot])

    def wait_idx(slot):
        pltpu.make_async_copy(rel_idx.at[pl.ds(0, SEQ)],
                              idx2.at[slot, pl.ds(0, SEQ)], sem_idx.at[slot]).wait()
        pltpu.make_async_copy(pos_idx.at[pl.ds(0, SEQ)],
                              pidx2.at[slot, pl.ds(0, SEQ)], sem_idx.at[slot]).wait()

    def fire_gather(slot):
        for o in range(0, SEQ, GW):
            pltpu.async_copy(rel_tab.at[idx2.at[slot, pl.ds(o, GW)]],
                             rows2.at[slot, pl.ds(o, GW)], sem_g.at[slot])

    def wait_gather(slot):
        for o in range(0, SEQ, GW):
            pltpu.make_async_copy(rel_tab.at[idx2.at[slot, pl.ds(o, GW)]],
                                  rows2.at[slot, pl.ds(o, GW)], sem_g.at[slot]).wait()

    def fire_out(i, slot):
        pltpu.async_copy(rows2.at[slot], out.at[b0 + i], sem_out.at[slot])

    def wait_out(i, slot):
        pltpu.make_async_copy(rows2.at[slot], out.at[b0 + i], sem_out.at[slot]).wait()

    pltpu.sync_copy(pos_tab, pos_v)

    fire_idx(0, 0)
    fire_idx(1, 1)
    fire_idx(2, 2)
    wait_idx(0)
    fire_gather(0)
    wait_idx(1)
    fire_gather(1)

    iota = lax.iota(jnp.int32, LANES)

    def step(i, carry):
        s = lax.rem(i, NBUF)
        wait_gather(s)

        @pl.when(i + 3 < b_per_w)
        def _():
            fire_idx(i + 3, lax.rem(i + 3, NBUF))

        @pl.when(i + 2 < b_per_w)
        def _():
            s2 = lax.rem(i + 2, NBUF)

            @pl.when(i >= 2)
            def _():
                wait_out(i - 2, s2)

            wait_idx(s2)
            fire_gather(s2)

        rr = rows2.at[s]
        pp = pidx2.at[s]

        def grp(g, gcarry):
            r0 = g * LANES
            pv = pp[pl.ds(r0, LANES)]
            for j in range(LANES):
                pb = jnp.full((LANES,), pv[j], jnp.int32)
                r = r0 + j
                for c in range(EMBED // LANES):
                    v = plsc.load_gather(pos_v, [pb, iota + (c * LANES)])
                    plsc.addupdate(rr.at[r, pl.ds(c * LANES, LANES)], v)
            return gcarry

        lax.fori_loop(0, SEQ // LANES, grp, 0)

        pvt = pp[pl.ds(SEQ - LANES, LANES)]
        for j in range(LANES - SEQ % LANES, LANES):
            pb = jnp.full((LANES,), pvt[j], jnp.int32)
            r = SEQ - LANES + j
            for c in range(EMBED // LANES):
                v = plsc.load_gather(pos_v, [pb, iota + (c * LANES)])
                plsc.addupdate(rr.at[r, pl.ds(c * LANES, LANES)], v)

        fire_out(i, s)
        return carry

    lax.fori_loop(0, b_per_w, step, 0)
    for j in range(NBUF):
        i = b_per_w - NBUF + j
        wait_out(i, lax.rem(i, NBUF))


def kernel(rel_seq, pos_seq, rel_table, pos_table):
    b, s = rel_seq.shape
    n = b * s
    run = pl.kernel(
        _body,
        out_type=jax.ShapeDtypeStruct((b, s, EMBED), jnp.float32),
        mesh=_mesh,
        scratch_types=[
            pltpu.VMEM((POS_ROWS, EMBED), jnp.float32),
            pltpu.VMEM((NBUF, 256), jnp.int32),
            pltpu.VMEM((NBUF, 256), jnp.int32),
            pltpu.VMEM((NBUF, SEQ, EMBED), jnp.float32),
            pltpu.SemaphoreType.DMA((NBUF,)),
            pltpu.SemaphoreType.DMA((NBUF,)),
            pltpu.SemaphoreType.DMA((NBUF,)),
        ],
        compiler_params=pltpu.CompilerParams(
            use_tc_tiling_on_sc=False, needs_layout_passes=False
        ),
    )
    return run(rel_seq.reshape(n), pos_seq.reshape(n), rel_table, pos_table)


# speedup vs baseline: 2.1747x; 1.0002x over previous
import jax
import jax.numpy as jnp
from jax import lax
from jax.experimental import pallas as pl
from jax.experimental.pallas import tpu as pltpu
from jax.experimental.pallas import tpu_sc as plsc

EMBED = 64
LANES = 16
POS_ROWS = 512
NC, NS = 2, 16
NW = NC * NS
SEQ = 200
GW = 40
NBUF = 6

_mesh = plsc.VectorSubcoreMesh(
    core_axis_name="c", subcore_axis_name="s", num_cores=NC, num_subcores=NS
)


def _body(rel_idx, pos_idx, rel_tab, pos_tab, out,
          pos_v, idx2, pidx2, rows2, sem_idx, sem_g, sem_out):
    n = rel_idx.shape[0]
    nb = out.shape[0]
    b_per_w = nb // NW
    wid = lax.axis_index("s") * NC + lax.axis_index("c")
    b0 = wid * b_per_w

    def fire_idx(i, slot):
        base = (b0 + i) * SEQ
        pltpu.async_copy(rel_idx.at[pl.ds(base, SEQ)],
                         idx2.at[slot, pl.ds(0, SEQ)], sem_idx.at[slot])
        pltpu.async_copy(pos_idx.at[pl.ds(base, SEQ)],
                         pidx2.at[slot, pl.ds(0, SEQ)], sem_idx.at[slot])

    def wait_idx(slot):
        pltpu.make_async_copy(rel_idx.at[pl.ds(0, SEQ)],
                              idx2.at[slot, pl.ds(0, SEQ)], sem_idx.at[slot]).wait()
        pltpu.make_async_copy(pos_idx.at[pl.ds(0, SEQ)],
                              pidx2.at[slot, pl.ds(0, SEQ)], sem_idx.at[slot]).wait()

    def fire_gather(slot):
        for o in range(0, SEQ, GW):
            pltpu.async_copy(rel_tab.at[idx2.at[slot, pl.ds(o, GW)]],
                             rows2.at[slot, pl.ds(o, GW)], sem_g.at[slot])

    def wait_gather(slot):
        for o in range(0, SEQ, GW):
            pltpu.make_async_copy(rel_tab.at[idx2.at[slot, pl.ds(o, GW)]],
                                  rows2.at[slot, pl.ds(o, GW)], sem_g.at[slot]).wait()

    def fire_out(i, slot):
        pltpu.async_copy(rows2.at[slot], out.at[b0 + i], sem_out.at[slot])

    def wait_out(i, slot):
        pltpu.make_async_copy(rows2.at[slot], out.at[b0 + i], sem_out.at[slot]).wait()

    pltpu.sync_copy(pos_tab, pos_v)

    fire_idx(0, 0)
    fire_idx(1, 1)
    fire_idx(2, 2)
    fire_idx(3, 3)
    wait_idx(0)
    fire_gather(0)
    wait_idx(1)
    fire_gather(1)
    wait_idx(2)
    fire_gather(2)

    iota = lax.iota(jnp.int32, LANES)

    def step(i, carry):
        s = lax.rem(i, NBUF)
        wait_gather(s)

        @pl.when(i + 4 < b_per_w)
        def _():
            fire_idx(i + 4, lax.rem(i + 4, NBUF))

        @pl.when(i + 3 < b_per_w)
        def _():
            s2 = lax.rem(i + 3, NBUF)

            @pl.when(i >= 3)
            def _():
                wait_out(i - 3, s2)

            wait_idx(s2)
            fire_gather(s2)

        rr = rows2.at[s]
        pp = pidx2.at[s]

        def grp(g, gcarry):
            r0 = g * LANES
            pv = pp[pl.ds(r0, LANES)]
            for j in range(LANES):
                pb = jnp.full((LANES,), pv[j], jnp.int32)
                r = r0 + j
                for c in range(EMBED // LANES):
                    v = plsc.load_gather(pos_v, [pb, iota + (c * LANES)])
                    plsc.addupdate(rr.at[r, pl.ds(c * LANES, LANES)], v)
            return gcarry

        lax.fori_loop(0, SEQ // LANES, grp, 0)

        pvt = pp[pl.ds(SEQ - LANES, LANES)]
        for j in range(LANES - SEQ % LANES, LANES):
            pb = jnp.full((LANES,), pvt[j], jnp.int32)
            r = SEQ - LANES + j
            for c in range(EMBED // LANES):
                v = plsc.load_gather(pos_v, [pb, iota + (c * LANES)])
                plsc.addupdate(rr.at[r, pl.ds(c * LANES, LANES)], v)

        fire_out(i, s)
        return carry

    lax.fori_loop(0, b_per_w, step, 0)
    for j in range(NBUF):
        i = b_per_w - NBUF + j
        wait_out(i, lax.rem(i, NBUF))


def kernel(rel_seq, pos_seq, rel_table, pos_table):
    b, s = rel_seq.shape
    n = b * s
    run = pl.kernel(
        _body,
        out_type=jax.ShapeDtypeStruct((b, s, EMBED), jnp.float32),
        mesh=_mesh,
        scratch_types=[
            pltpu.VMEM((POS_ROWS, EMBED), jnp.float32),
            pltpu.VMEM((NBUF, 256), jnp.int32),
            pltpu.VMEM((NBUF, 256), jnp.int32),
            pltpu.VMEM((NBUF, SEQ, EMBED), jnp.float32),
            pltpu.SemaphoreType.DMA((NBUF,)),
            pltpu.SemaphoreType.DMA((NBUF,)),
            pltpu.SemaphoreType.DMA((NBUF,)),
        ],
        compiler_params=pltpu.CompilerParams(
            use_tc_tiling_on_sc=False, needs_layout_passes=False
        ),
    )
    return run(rel_seq.reshape(n), pos_seq.reshape(n), rel_table, pos_table)
